# Initial kernel scaffold; baseline (speedup 1.0000x reference)
import jax
import jax.numpy as jnp
from jax.experimental import pallas as pl


def kernel(x, adj_mat, W_self_0, b_self_0, W_comb_0, b_comb_0, W_self_1, b_self_1, W_comb_1, b_comb_1, W_self_2, b_self_2, W_comb_2, b_comb_2):
    raise NotImplementedError("write your pallas kernel here")


# speedup vs baseline: 148.4152x; 148.4152x over previous
import functools

import jax
import jax.numpy as jnp
from jax import lax
from jax.experimental import pallas as pl
from jax.experimental.pallas import tpu as pltpu
from jax.experimental.pallas import tpu_sc as plsc

N = 100000
DEG = 64
C = 3
V = 68
NG = DEG + 1
GP = 72

NC = 2
NS = 16

CHUNK_A = N // 10
BLK_A = 400
NBLK_A = CHUNK_A // BLK_A
GRP_PER_TEC = 13

_SC_PARAMS = pltpu.CompilerParams(needs_layout_passes=False)


def _vert_gather_sum(plane, adjfb, fb, i64):
    def jfn(j, acc):
        idxv = plsc.load_gather(adjfb, [i64 + (fb + j)])
        return acc + plsc.load_gather(plane, [idxv])

    return lax.fori_loop(0, DEG, jfn, jnp.zeros((16,), jnp.float32),
                         unroll=16)


def _sc_pool_dense_body(xT, adjf, pooled, plane, adjb, poolb, dsem):
    cid = lax.axis_index("c")
    sid = lax.axis_index("s")
    wid = sid * NC + cid

    @pl.when(wid < 30)
    def _():
        comp = wid // 10
        chunk = wid % 10
        pltpu.sync_copy(xT.at[pl.ds(comp * N, N)], plane)
        i64 = lax.iota(jnp.int32, 16) * DEG
        base = chunk * CHUNK_A

        def block_fn(kb, _):
            node0 = base + kb * BLK_A
            pltpu.async_copy(adjf.at[pl.ds(node0 * DEG, BLK_A * DEG)],
                             adjb, dsem).wait()

            def grp_fn(t, _):
                r0 = t * 16
                acc = _vert_gather_sum(plane, adjb, r0 * DEG, i64)
                poolb[pl.ds(r0, 16)] = acc * (1.0 / DEG)
                return 0

            lax.fori_loop(0, BLK_A // 16, grp_fn, 0)
            pltpu.sync_copy(poolb, pooled.at[pl.ds(comp * N + node0, BLK_A)])
            return 0

        lax.fori_loop(0, NBLK_A, block_fn, 0)


def _sc_pool_dense(xT, adjf):
    return pl.kernel(
        _sc_pool_dense_body,
        out_type=jax.ShapeDtypeStruct((C * N,), jnp.float32),
        mesh=plsc.VectorSubcoreMesh(core_axis_name="c", subcore_axis_name="s"),
        compiler_params=_SC_PARAMS,
        scratch_types=[
            pltpu.VMEM((N,), jnp.float32),
            pltpu.VMEM((BLK_A * DEG,), jnp.int32),
            pltpu.VMEM((BLK_A,), jnp.float32),
            pltpu.SemaphoreType.DMA,
        ],
    )(xT, adjf)


_WINDOWS = ((0, 0), (16, 0), (32, 0), (48, 0), (56, 8))


def _sc_pool_sparse_body(x1T, adjf, s3, pooled2, x1s3, plane, s3b, rowb,
                         poolb, valb, dsem):
    cid = lax.axis_index("c")
    sid = lax.axis_index("s")
    wid = sid * NC + cid

    @pl.when(wid < 15)
    def _():
        comp = wid // 5
        chunk = wid % 5
        pltpu.sync_copy(x1T.at[pl.ds(comp * N, N)], plane)
        i64 = lax.iota(jnp.int32, 16) * DEG

        def grp_fn(gi, _):
            g = chunk * GRP_PER_TEC + gi
            pltpu.sync_copy(s3.at[pl.ds(g * GP, GP)], s3b)
            copies = []
            for off, l0 in _WINDOWS:
                wvec = s3b[pl.ds(off, 16)]
                for l in range(l0, 16):
                    p = off + l
                    copies.append(pltpu.async_copy(
                        adjf.at[pl.ds(wvec[l] * DEG, DEG)],
                        rowb.at[pl.ds(p * DEG, DEG)], dsem))
            for cp in copies:
                cp.wait()
            for off, _unused in _WINDOWS:
                acc = _vert_gather_sum(plane, rowb, off * DEG, i64)
                poolb[pl.ds(off, 16)] = acc * (1.0 / DEG)
                valb[pl.ds(off, 16)] = plsc.load_gather(
                    plane, [s3b[pl.ds(off, 16)]])
            out0 = (comp * NG + g) * GP
            pltpu.sync_copy(poolb, pooled2.at[pl.ds(out0, GP)])
            pltpu.sync_copy(valb, x1s3.at[pl.ds(out0, GP)])
            return 0

        lax.fori_loop(0, GRP_PER_TEC, grp_fn, 0)


def _sc_pool_sparse(x1T, adjf, s3):
    return pl.kernel(
        _sc_pool_sparse_body,
        out_type=(
            jax.ShapeDtypeStruct((C * NG * GP,), jnp.float32),
            jax.ShapeDtypeStruct((C * NG * GP,), jnp.float32),
        ),
        mesh=plsc.VectorSubcoreMesh(core_axis_name="c", subcore_axis_name="s"),
        compiler_params=_SC_PARAMS,
        scratch_types=[
            pltpu.VMEM((N,), jnp.float32),
            pltpu.VMEM((GP,), jnp.int32),
            pltpu.VMEM((GP * DEG,), jnp.int32),
            pltpu.VMEM((GP,), jnp.float32),
            pltpu.VMEM((GP,), jnp.float32),
            pltpu.SemaphoreType.DMA,
        ],
    )(x1T, adjf, s3)


def _combine_rows(xrows, prows, ws, bs, wc, bc):
    h = [jnp.tanh(ws[c, 0] * xrows[0] + ws[c, 1] * xrows[1]
                  + ws[c, 2] * xrows[2] + bs[c]) for c in range(C)]
    out = []
    for c in range(C):
        acc = bc[c]
        for k in range(C):
            acc = acc + wc[c, k, 0] * h[k] + wc[c, k, 1] * prows[k]
        out.append(jnp.tanh(jnp.tanh(acc)))
    return out


def _tc_combine_body(ws_ref, bs_ref, wc_ref, bc_ref, xT_ref, pT_ref, out_ref):
    xrows = [xT_ref[c:c + 1, :] for c in range(C)]
    prows = [pT_ref[c:c + 1, :] for c in range(C)]
    out = _combine_rows(xrows, prows, ws_ref, bs_ref, wc_ref, bc_ref)
    out_ref[...] = jnp.concatenate(out, axis=0)


def _tc_combine(xT, pT, ws, bs, wc, bc):
    smem = pl.BlockSpec(memory_space=pltpu.SMEM)
    return pl.pallas_call(
        _tc_combine_body,
        out_shape=jax.ShapeDtypeStruct((C, N), jnp.float32),
        in_specs=[smem, smem, smem, smem, pl.BlockSpec(), pl.BlockSpec()],
    )(ws, bs, wc, bc, xT, pT)


def _tc_finish_body(ws1_ref, bs1_ref, wc1_ref, bc1_ref,
                    ws2_ref, bs2_ref, wc2_ref, bc2_ref,
                    x1_ref, p2_ref, out_ref):
    x1 = [x1_ref[c] for c in range(C)]
    p2 = [p2_ref[c] for c in range(C)]
    x2 = _combine_rows(x1, p2, ws1_ref, bs1_ref, wc1_ref, bc1_ref)
    selfs = [x2[c][0:1, :] for c in range(C)]
    pool3 = [jnp.sum(x2[c][1:NG, :], axis=0, keepdims=True) * (1.0 / DEG)
             for c in range(C)]
    out = _combine_rows(selfs, pool3, ws2_ref, bs2_ref, wc2_ref, bc2_ref)
    out_ref[...] = jnp.concatenate(out, axis=0)


def _tc_finish(x1s3, p2, ws1, bs1, wc1, bc1, ws2, bs2, wc2, bc2):
    smem = pl.BlockSpec(memory_space=pltpu.SMEM)
    return pl.pallas_call(
        _tc_finish_body,
        out_shape=jax.ShapeDtypeStruct((C, GP), jnp.float32),
        in_specs=[smem] * 8 + [pl.BlockSpec(), pl.BlockSpec()],
    )(ws1, bs1, wc1, bc1, ws2, bs2, wc2, bc2, x1s3, p2)


@jax.jit
def kernel(x, adj_mat,
           W_self_0, b_self_0, W_comb_0, b_comb_0,
           W_self_1, b_self_1, W_comb_1, b_comb_1,
           W_self_2, b_self_2, W_comb_2, b_comb_2):
    xT = x.T
    xTf = xT.reshape(C * N)
    adjf = adj_mat.reshape(N * DEG)

    last = jnp.arange(N - V, N, dtype=jnp.int32)
    s3 = jnp.zeros((NG, GP), jnp.int32)
    s3 = s3.at[0, :V].set(last)
    s3 = s3.at[1:, :V].set(adj_mat[N - V:, :].T)

    p1T = _sc_pool_dense(xTf, adjf).reshape(C, N)
    x1T = _tc_combine(xT, p1T, W_self_0, b_self_0, W_comb_0, b_comb_0)
    p2, x1s3 = _sc_pool_sparse(x1T.reshape(C * N), adjf, s3.reshape(-1))
    out = _tc_finish(x1s3.reshape(C, NG, GP), p2.reshape(C, NG, GP),
                     W_self_1, b_self_1, W_comb_1, b_comb_1,
                     W_self_2, b_self_2, W_comb_2, b_comb_2)
    return out[:, :V].T[:, :, None]


# speedup vs baseline: 162.6594x; 1.0960x over previous
import functools

import jax
import jax.numpy as jnp
from jax import lax
from jax.experimental import pallas as pl
from jax.experimental.pallas import tpu as pltpu
from jax.experimental.pallas import tpu_sc as plsc

N = 100000
DEG = 64
C = 3
V = 68
NG = DEG + 1
GP = 72

NC = 2
NS = 16

CHUNK_A = N // 10
BLK_A = 80
NBLK_A = CHUNK_A // BLK_A
GRP_PER_TEC = 13

_SC_PARAMS = pltpu.CompilerParams(needs_layout_passes=False)


def _vert_gather_sum(plane, adjfb, fb, i64):
    accs = [jnp.zeros((16,), jnp.float32) for _ in range(4)]
    for j in range(DEG):
        idxv = plsc.load_gather(adjfb, [i64 + (fb + j)])
        accs[j % 4] = accs[j % 4] + plsc.load_gather(plane, [idxv])
    return (accs[0] + accs[1]) + (accs[2] + accs[3])


def _sc_pool_dense_body(xT, adjf, pooled, plane, adjb0, adjb1, poolb0, poolb1,
                        isem0, isem1, osem0, osem1):
    cid = lax.axis_index("c")
    sid = lax.axis_index("s")
    wid = sid * NC + cid

    @pl.when(wid < 30)
    def _():
        comp = wid // 10
        chunk = wid % 10
        pltpu.sync_copy(xT.at[pl.ds(comp * N, N)], plane)
        i64 = lax.iota(jnp.int32, 16) * DEG
        base = chunk * CHUNK_A

        def in_slice(blk):
            return adjf.at[pl.ds((base + blk * BLK_A) * DEG, BLK_A * DEG)]

        def out_slice(blk):
            return pooled.at[pl.ds(comp * N + base + blk * BLK_A, BLK_A)]

        bufs = ((adjb0, poolb0, isem0, osem0), (adjb1, poolb1, isem1, osem1))
        pltpu.async_copy(in_slice(0), adjb0, isem0)

        def round_fn(kb, _):
            for b, (adjb, poolb, isem, osem) in enumerate(bufs):
                blk = kb * 2 + b

                @pl.when(blk < NBLK_A)
                def _():
                    @pl.when(blk + 1 < NBLK_A)
                    def _():
                        nxt = bufs[1 - b]
                        pltpu.async_copy(in_slice(blk + 1), nxt[0], nxt[2])

                    pltpu.make_async_copy(in_slice(blk), adjb, isem).wait()

                    @pl.when(blk >= 2)
                    def _():
                        pltpu.make_async_copy(poolb, out_slice(blk - 2),
                                              osem).wait()

                    def grp_fn(t, _):
                        r0 = t * 16
                        acc = _vert_gather_sum(plane, adjb, r0 * DEG, i64)
                        poolb[pl.ds(r0, 16)] = acc * (1.0 / DEG)
                        return 0

                    lax.fori_loop(0, BLK_A // 16, grp_fn, 0)
                    pltpu.async_copy(poolb, out_slice(blk), osem)
            return 0

        lax.fori_loop(0, (NBLK_A + 1) // 2, round_fn, 0)
        pltpu.make_async_copy(poolb1, out_slice(NBLK_A - 2), osem1).wait()
        pltpu.make_async_copy(poolb0, out_slice(NBLK_A - 1), osem0).wait()


def _sc_pool_dense(xT, adjf):
    return pl.kernel(
        _sc_pool_dense_body,
        out_type=jax.ShapeDtypeStruct((C * N,), jnp.float32),
        mesh=plsc.VectorSubcoreMesh(core_axis_name="c", subcore_axis_name="s"),
        compiler_params=_SC_PARAMS,
        scratch_types=[
            pltpu.VMEM((N,), jnp.float32),
            pltpu.VMEM((BLK_A * DEG,), jnp.int32),
            pltpu.VMEM((BLK_A * DEG,), jnp.int32),
            pltpu.VMEM((BLK_A,), jnp.float32),
            pltpu.VMEM((BLK_A,), jnp.float32),
            pltpu.SemaphoreType.DMA,
            pltpu.SemaphoreType.DMA,
            pltpu.SemaphoreType.DMA,
            pltpu.SemaphoreType.DMA,
        ],
    )(xT, adjf)


_WINDOWS = ((0, 0), (16, 0), (32, 0), (48, 0), (56, 8))


def _sc_pool_sparse_body(x1T, adjf, s3, pooled2, x1s3, plane, s3b, rowb,
                         poolb, valb, dsem):
    cid = lax.axis_index("c")
    sid = lax.axis_index("s")
    wid = sid * NC + cid

    @pl.when(wid < 15)
    def _():
        comp = wid // 5
        chunk = wid % 5
        pltpu.sync_copy(x1T.at[pl.ds(comp * N, N)], plane)
        i64 = lax.iota(jnp.int32, 16) * DEG

        def grp_fn(gi, _):
            g = chunk * GRP_PER_TEC + gi
            pltpu.sync_copy(s3.at[pl.ds(g * GP, GP)], s3b)
            copies = []
            for off, l0 in _WINDOWS:
                wvec = s3b[pl.ds(off, 16)]
                for l in range(l0, 16):
                    p = off + l
                    copies.append(pltpu.async_copy(
                        adjf.at[pl.ds(wvec[l] * DEG, DEG)],
                        rowb.at[pl.ds(p * DEG, DEG)], dsem))
            for cp in copies:
                cp.wait()
            for off, _unused in _WINDOWS:
                acc = _vert_gather_sum(plane, rowb, off * DEG, i64)
                poolb[pl.ds(off, 16)] = acc * (1.0 / DEG)
                valb[pl.ds(off, 16)] = plsc.load_gather(
                    plane, [s3b[pl.ds(off, 16)]])
            out0 = (comp * NG + g) * GP
            pltpu.sync_copy(poolb, pooled2.at[pl.ds(out0, GP)])
            pltpu.sync_copy(valb, x1s3.at[pl.ds(out0, GP)])
            return 0

        lax.fori_loop(0, GRP_PER_TEC, grp_fn, 0)


def _sc_pool_sparse(x1T, adjf, s3):
    return pl.kernel(
        _sc_pool_sparse_body,
        out_type=(
            jax.ShapeDtypeStruct((C * NG * GP,), jnp.float32),
            jax.ShapeDtypeStruct((C * NG * GP,), jnp.float32),
        ),
        mesh=plsc.VectorSubcoreMesh(core_axis_name="c", subcore_axis_name="s"),
        compiler_params=_SC_PARAMS,
        scratch_types=[
            pltpu.VMEM((N,), jnp.float32),
            pltpu.VMEM((GP,), jnp.int32),
            pltpu.VMEM((GP * DEG,), jnp.int32),
            pltpu.VMEM((GP,), jnp.float32),
            pltpu.VMEM((GP,), jnp.float32),
            pltpu.SemaphoreType.DMA,
        ],
    )(x1T, adjf, s3)


def _combine_rows(xrows, prows, ws, bs, wc, bc):
    h = [jnp.tanh(ws[c, 0] * xrows[0] + ws[c, 1] * xrows[1]
                  + ws[c, 2] * xrows[2] + bs[c]) for c in range(C)]
    out = []
    for c in range(C):
        acc = bc[c]
        for k in range(C):
            acc = acc + wc[c, k, 0] * h[k] + wc[c, k, 1] * prows[k]
        out.append(jnp.tanh(jnp.tanh(acc)))
    return out


def _tc_combine_body(ws_ref, bs_ref, wc_ref, bc_ref, xT_ref, pT_ref, out_ref):
    xrows = [xT_ref[c:c + 1, :] for c in range(C)]
    prows = [pT_ref[c:c + 1, :] for c in range(C)]
    out = _combine_rows(xrows, prows, ws_ref, bs_ref, wc_ref, bc_ref)
    out_ref[...] = jnp.concatenate(out, axis=0)


def _tc_combine(xT, pT, ws, bs, wc, bc):
    smem = pl.BlockSpec(memory_space=pltpu.SMEM)
    return pl.pallas_call(
        _tc_combine_body,
        out_shape=jax.ShapeDtypeStruct((C, N), jnp.float32),
        in_specs=[smem, smem, smem, smem, pl.BlockSpec(), pl.BlockSpec()],
    )(ws, bs, wc, bc, xT, pT)


def _tc_finish_body(ws1_ref, bs1_ref, wc1_ref, bc1_ref,
                    ws2_ref, bs2_ref, wc2_ref, bc2_ref,
                    x1_ref, p2_ref, out_ref):
    x1 = [x1_ref[c] for c in range(C)]
    p2 = [p2_ref[c] for c in range(C)]
    x2 = _combine_rows(x1, p2, ws1_ref, bs1_ref, wc1_ref, bc1_ref)
    selfs = [x2[c][0:1, :] for c in range(C)]
    pool3 = [jnp.sum(x2[c][1:NG, :], axis=0, keepdims=True) * (1.0 / DEG)
             for c in range(C)]
    out = _combine_rows(selfs, pool3, ws2_ref, bs2_ref, wc2_ref, bc2_ref)
    out_ref[...] = jnp.concatenate(out, axis=0)


def _tc_finish(x1s3, p2, ws1, bs1, wc1, bc1, ws2, bs2, wc2, bc2):
    smem = pl.BlockSpec(memory_space=pltpu.SMEM)
    return pl.pallas_call(
        _tc_finish_body,
        out_shape=jax.ShapeDtypeStruct((C, GP), jnp.float32),
        in_specs=[smem] * 8 + [pl.BlockSpec(), pl.BlockSpec()],
    )(ws1, bs1, wc1, bc1, ws2, bs2, wc2, bc2, x1s3, p2)


@jax.jit
def kernel(x, adj_mat,
           W_self_0, b_self_0, W_comb_0, b_comb_0,
           W_self_1, b_self_1, W_comb_1, b_comb_1,
           W_self_2, b_self_2, W_comb_2, b_comb_2):
    xT = x.T
    xTf = xT.reshape(C * N)
    adjf = adj_mat.reshape(N * DEG)

    last = jnp.arange(N - V, N, dtype=jnp.int32)
    s3 = jnp.zeros((NG, GP), jnp.int32)
    s3 = s3.at[0, :V].set(last)
    s3 = s3.at[1:, :V].set(adj_mat[N - V:, :].T)

    p1T = _sc_pool_dense(xTf, adjf).reshape(C, N)
    x1T = _tc_combine(xT, p1T, W_self_0, b_self_0, W_comb_0, b_comb_0)
    p2, x1s3 = _sc_pool_sparse(x1T.reshape(C * N), adjf, s3.reshape(-1))
    out = _tc_finish(x1s3.reshape(C, NG, GP), p2.reshape(C, NG, GP),
                     W_self_1, b_self_1, W_comb_1, b_comb_1,
                     W_self_2, b_self_2, W_comb_2, b_comb_2)
    return out[:, :V].T[:, :, None]
